# baseline (speedup 1.0000x reference)
import jax
import jax.numpy as jnp
from jax import lax
from jax.experimental import pallas as pl
from jax.experimental.pallas import tpu as pltpu
from jax.experimental.pallas import tpu_sc as plsc

VOCAB = 1000000
D = 128
SEQ = 200
BATCH = 4096
NC, NS = 2, 16
NW = NC * NS
ROWS = BATCH * SEQ
SPW = BATCH // NW
CHUNK = 64
NCHUNK = 2 * SEQ
NBUF = 5
LEAD = 3


def _embed_body(tok_hbm, table_hbm, pos_hbm, rtab_hbm, out_hbm,
                idx_v, pos_v, rtab_v, rows_v, oidx_v,
                g0, g1, g2, g3, g4, s0, s1, s2, s3, s4):
    gsems = (g0, g1, g2, g3, g4)
    ssems = (s0, s1, s2, s3, s4)
    wid = lax.axis_index("s") * NC + lax.axis_index("c")
    wbase = wid * SPW * SEQ
    pltpu.sync_copy(tok_hbm.at[wid], idx_v)
    pltpu.sync_copy(pos_hbm, pos_v)
    pltpu.sync_copy(rtab_hbm, rtab_v)

    for b in range(LEAD):
        pltpu.async_copy(table_hbm.at[idx_v.at[b]], rows_v.at[b], gsems[b])

    def process(i, carry):
        for b in range(NBUF):
            c = i * NBUF + b
            l = c // 2
            pltpu.make_async_copy(
                table_hbm.at[idx_v.at[c]], rows_v.at[b], gsems[b]).wait()
            off = wbase + lax.rem(c, 2) * (CHUNK * SEQ) + l
            for v in range(CHUNK // 16):
                sl = pl.ds(v * 16, 16)
                oidx_v[b, sl] = rtab_v[sl] + off
            rb = rows_v.at[b]
            pv = [pos_v[pl.ds(l * D + v * 16, 16)] for v in range(D // 16)]

            @plsc.parallel_loop(0, CHUNK, unroll=4)
            def row_body(r):
                for v in range(D // 16):
                    sl = pl.ds(v * 16, 16)
                    rb[r, sl] = rb[r, sl] + pv[v]

            pltpu.async_copy(rows_v.at[b], out_hbm.at[oidx_v.at[b]], ssems[b])
            nb = (b + LEAD) % NBUF
            cn = c + LEAD

            @pl.when(cn < NCHUNK)
            def _():
                @pl.when(cn >= NBUF)
                def _():
                    pltpu.make_async_copy(
                        rows_v.at[nb], out_hbm.at[oidx_v.at[nb]],
                        ssems[nb]).wait()
                pltpu.async_copy(table_hbm.at[idx_v.at[cn]], rows_v.at[nb],
                                 gsems[nb])
        return carry

    lax.fori_loop(0, NCHUNK // NBUF, process, 0)

    for b in range(NBUF):
        pltpu.make_async_copy(
            rows_v.at[b], out_hbm.at[oidx_v.at[b]], ssems[b]).wait()


@jax.jit
def _embed(tok_r, table, pos_flat, rtab):
    mesh = plsc.VectorSubcoreMesh(core_axis_name="c", subcore_axis_name="s")
    f = pl.kernel(
        _embed_body,
        out_type=jax.ShapeDtypeStruct((ROWS, D), jnp.float32),
        mesh=mesh,
        scratch_types=[
            pltpu.VMEM((NCHUNK, CHUNK), jnp.int32),
            pltpu.VMEM((SEQ * D,), jnp.float32),
            pltpu.VMEM((CHUNK,), jnp.int32),
            pltpu.VMEM((NBUF, CHUNK, D), jnp.float32),
            pltpu.VMEM((NBUF, CHUNK), jnp.int32),
        ] + [pltpu.SemaphoreType.DMA] * (2 * NBUF),
    )
    return f(tok_r, table, pos_flat, rtab)


def kernel(token_ids, table, pos_encoding):
    tok_r = (token_ids.reshape(NW, SPW, SEQ).transpose(0, 2, 1)
             .astype(jnp.int32).reshape(NW, NCHUNK, CHUNK))
    pos_flat = pos_encoding.reshape(SEQ * D).astype(jnp.float32)
    rtab = jnp.arange(CHUNK, dtype=jnp.int32) * SEQ
    out = _embed(tok_r, table, pos_flat, rtab)
    return out.reshape(BATCH, SEQ, D)


# speedup vs baseline: 1.0291x; 1.0291x over previous
import jax
import jax.numpy as jnp
from jax import lax
from jax.experimental import pallas as pl
from jax.experimental.pallas import tpu as pltpu
from jax.experimental.pallas import tpu_sc as plsc

VOCAB = 1000000
D = 128
SEQ = 200
BATCH = 4096
NC, NS = 2, 16
NW = NC * NS
ROWS = BATCH * SEQ
SPW = BATCH // NW
NBUF = 4
LEAD = 2


def _embed_body(tok_hbm, table_hbm, pos_hbm, rtab_hbm, out_hbm,
                idx_v, pos_v, rtab_v, rows_v, oidx_v,
                g0, g1, g2, g3, s0, s1, s2, s3):
    gsems = (g0, g1, g2, g3)
    ssems = (s0, s1, s2, s3)
    wid = lax.axis_index("s") * NC + lax.axis_index("c")
    wbase = wid * SPW * SEQ
    pltpu.sync_copy(tok_hbm.at[wid], idx_v)

    for b in range(LEAD):
        pltpu.async_copy(table_hbm.at[idx_v.at[b]], rows_v.at[b], gsems[b])

    pltpu.sync_copy(pos_hbm, pos_v)
    pltpu.sync_copy(rtab_hbm, rtab_v)

    def process(i, carry):
        for b in range(NBUF):
            l = i * NBUF + b
            pltpu.make_async_copy(
                table_hbm.at[idx_v.at[l]], rows_v.at[b], gsems[b]).wait()
            off = wbase + l
            for v in range(SPW // 16):
                sl = pl.ds(v * 16, 16)
                oidx_v[b, sl] = rtab_v[sl] + off
            rb = rows_v.at[b]
            pv = [pos_v[pl.ds(l * D + v * 16, 16)] for v in range(D // 16)]

            @plsc.parallel_loop(0, SPW, unroll=4)
            def row_body(r):
                for v in range(D // 16):
                    sl = pl.ds(v * 16, 16)
                    rb[r, sl] = rb[r, sl] + pv[v]

            pltpu.async_copy(rows_v.at[b], out_hbm.at[oidx_v.at[b]], ssems[b])
            nb = (b + LEAD) % NBUF
            ln = l + LEAD

            @pl.when(ln < SEQ)
            def _():
                @pl.when(ln >= NBUF)
                def _():
                    pltpu.make_async_copy(
                        rows_v.at[nb], out_hbm.at[oidx_v.at[nb]],
                        ssems[nb]).wait()
                pltpu.async_copy(table_hbm.at[idx_v.at[ln]], rows_v.at[nb],
                                 gsems[nb])
        return carry

    lax.fori_loop(0, SEQ // NBUF, process, 0)

    for b in range(NBUF):
        pltpu.make_async_copy(
            rows_v.at[b], out_hbm.at[oidx_v.at[b]], ssems[b]).wait()


@jax.jit
def _embed(tok_r, table, pos_flat, rtab):
    mesh = plsc.VectorSubcoreMesh(core_axis_name="c", subcore_axis_name="s")
    f = pl.kernel(
        _embed_body,
        out_type=jax.ShapeDtypeStruct((ROWS, D), jnp.float32),
        mesh=mesh,
        scratch_types=[
            pltpu.VMEM((SEQ, SPW), jnp.int32),
            pltpu.VMEM((SEQ * D,), jnp.float32),
            pltpu.VMEM((SPW,), jnp.int32),
            pltpu.VMEM((NBUF, SPW, D), jnp.float32),
            pltpu.VMEM((NBUF, SPW), jnp.int32),
        ] + [pltpu.SemaphoreType.DMA] * (2 * NBUF),
    )
    return f(tok_r, table, pos_flat, rtab)


def kernel(token_ids, table, pos_encoding):
    tok_r = (token_ids.reshape(NW, SPW, SEQ).transpose(0, 2, 1)
             .astype(jnp.int32))
    pos_flat = pos_encoding.reshape(SEQ * D).astype(jnp.float32)
    rtab = jnp.arange(SPW, dtype=jnp.int32) * SEQ
    out = _embed(tok_r, table, pos_flat, rtab)
    return out.reshape(BATCH, SEQ, D)
